# baseline (speedup 1.0000x reference)
import functools

import jax
import jax.numpy as jnp
from jax import lax
from jax.experimental import pallas as pl
from jax.experimental.pallas import tpu as pltpu
from jax.experimental.pallas import tpu_sc as plsc

_T, _N, _C, _DO, _DI = 32, 16, 2, 512, 64
_M = _N * _C * _DO * _DI
_NW = 32
_COLS_W = _M // _NW
_MC = 1024
_NCHUNK = _COLS_W // _MC
_G = _MC // 16

_mesh = plsc.VectorSubcoreMesh(core_axis_name="c", subcore_axis_name="s")


def _sigmoid(v):
    return 1.0 / (1.0 + jnp.exp(-v))


@functools.partial(
    pl.kernel,
    mesh=_mesh,
    out_type=jax.ShapeDtypeStruct((_T, _M), jnp.float32),
    scratch_types=[
        pltpu.VMEM((_T, _MC), jnp.float32),
        pltpu.VMEM((_T, _MC), jnp.float32),
        pltpu.VMEM((_MC,), jnp.float32),
        pltpu.SemaphoreType.DMA,
        pltpu.SemaphoreType.DMA,
    ],
    compiler_params=pltpu.CompilerParams(
        use_tc_tiling_on_sc=True, needs_layout_passes=False
    ),
)
def _delay_sc(x_hbm, br0_hbm, out_hbm, in_sp, out_sp, br_sp, sem_in, sem_out):
    wid = lax.axis_index("s") * 2 + lax.axis_index("c")
    base = wid * _COLS_W

    def chunk_body(ci, carry):
        m0 = base + ci * _MC

        copies = [
            pltpu.make_async_copy(
                x_hbm.at[t, pl.ds(m0, _MC)], in_sp.at[t], sem_in
            )
            for t in range(_T)
        ]
        cbr = pltpu.make_async_copy(br0_hbm.at[pl.ds(m0, _MC)], br_sp, sem_in)
        for cp in copies:
            cp.start()
        cbr.start()
        for cp in copies:
            cp.wait()
        cbr.wait()

        def group_body(j, c2):
            joff = j * 16
            col = joff + lax.iota(jnp.int32, 16)

            s0 = _sigmoid(in_sp[0, pl.ds(joff, 16)])
            in_sp[0, pl.ds(joff, 16)] = s0
            mx = s0
            am = jnp.zeros((16,), jnp.int32)
            for t in range(1, _T):
                st = _sigmoid(in_sp[t, pl.ds(joff, 16)])
                in_sp[t, pl.ds(joff, 16)] = st
                gt = st > mx
                am = jnp.where(gt, t, am)
                mx = jnp.where(gt, st, mx)

            brf = jnp.minimum(
                br_sp[pl.ds(joff, 16)], (31 - am).astype(jnp.float32)
            )
            br = brf.astype(jnp.int32)
            for t in range(_T):
                r = (t - br) & 31
                out_sp[t, pl.ds(joff, 16)] = plsc.load_gather(in_sp, [r, col])
            return c2

        lax.fori_loop(0, _G, group_body, 0, unroll=False)

        ocopies = [
            pltpu.make_async_copy(
                out_sp.at[t], out_hbm.at[t, pl.ds(m0, _MC)], sem_out
            )
            for t in range(_T)
        ]
        for cp in ocopies:
            cp.start()
        for cp in ocopies:
            cp.wait()
        return carry

    lax.fori_loop(0, _NCHUNK, chunk_body, 0, unroll=False)


def kernel(input, delay):
    x = input.reshape(_T, _M)
    bd = jnp.broadcast_to(delay[None, None, :, :], (_N, _C, _DO, _DI))
    bf = jnp.floor(bd)
    bern = jax.random.bernoulli(jax.random.key(1), bd - bf)
    br0 = jnp.where(bern, bf + 1.0, bf).reshape(_M)
    out = _delay_sc(x, br0)
    return out.reshape(_T, _N, _C, _DO, _DI)


# speedup vs baseline: 1.2311x; 1.2311x over previous
import functools

import jax
import jax.numpy as jnp
from jax import lax
from jax.experimental import pallas as pl
from jax.experimental.pallas import tpu as pltpu
from jax.experimental.pallas import tpu_sc as plsc

_T, _N, _C, _DO, _DI = 32, 16, 2, 512, 64
_M = _N * _C * _DO * _DI
_NW = 32
_COLS_W = _M // _NW
_NO = 16
_MC = _NO * _DI
_NCHUNK = _DO // _NO
_G = _MC // 16

_mesh = plsc.VectorSubcoreMesh(core_axis_name="c", subcore_axis_name="s")


def _sigmoid(v):
    return 1.0 / (1.0 + jnp.exp(-v))


@functools.partial(
    pl.kernel,
    mesh=_mesh,
    out_type=jax.ShapeDtypeStruct((_T, _N, _C, _DO, _DI), jnp.float32),
    scratch_types=[
        pltpu.VMEM((_T, _NO, _DI), jnp.float32),
        pltpu.VMEM((_T, _NO, _DI), jnp.float32),
        pltpu.VMEM((_MC,), jnp.float32),
        pltpu.SemaphoreType.DMA,
        pltpu.SemaphoreType.DMA,
    ],
    compiler_params=pltpu.CompilerParams(
        use_tc_tiling_on_sc=False, needs_layout_passes=False
    ),
)
def _delay_sc(x_hbm, br0_hbm, out_hbm, in_sp, out_sp, br_sp, sem_in, sem_out):
    wid = lax.axis_index("s") * 2 + lax.axis_index("c")
    n0 = wid // _C
    c0 = wid % _C

    def chunk_body(ci, carry):
        o0 = ci * _NO
        m0 = wid * _COLS_W + ci * _MC

        copies = [
            pltpu.make_async_copy(
                x_hbm.at[t, n0, c0, pl.ds(o0, _NO), :], in_sp.at[t], sem_in
            )
            for t in range(_T)
        ]
        cbr = pltpu.make_async_copy(br0_hbm.at[pl.ds(m0, _MC)], br_sp, sem_in)
        for cp in copies:
            cp.start()
        cbr.start()
        for cp in copies:
            cp.wait()
        cbr.wait()

        def group_body(j, c2):
            osub = j >> 2
            i0 = (j & 3) << 4
            ivec = i0 + lax.iota(jnp.int32, 16)
            ovec = jnp.full((16,), osub, jnp.int32)

            s0 = _sigmoid(in_sp[0, osub, pl.ds(i0, 16)])
            in_sp[0, osub, pl.ds(i0, 16)] = s0
            mx = s0
            am = jnp.zeros((16,), jnp.int32)
            for t in range(1, _T):
                st = _sigmoid(in_sp[t, osub, pl.ds(i0, 16)])
                in_sp[t, osub, pl.ds(i0, 16)] = st
                gt = st > mx
                am = jnp.where(gt, t, am)
                mx = jnp.where(gt, st, mx)

            brf = jnp.minimum(
                br_sp[pl.ds(j * 16, 16)], (31 - am).astype(jnp.float32)
            )
            br = brf.astype(jnp.int32)
            for t in range(_T):
                r = (t - br) & 31
                out_sp[t, osub, pl.ds(i0, 16)] = plsc.load_gather(
                    in_sp, [r, ovec, ivec]
                )
            return c2

        lax.fori_loop(0, _G, group_body, 0, unroll=False)

        ocopies = [
            pltpu.make_async_copy(
                out_sp.at[t], out_hbm.at[t, n0, c0, pl.ds(o0, _NO), :], sem_out
            )
            for t in range(_T)
        ]
        for cp in ocopies:
            cp.start()
        for cp in ocopies:
            cp.wait()
        return carry

    lax.fori_loop(0, _NCHUNK, chunk_body, 0, unroll=False)


def kernel(input, delay):
    bd = jnp.broadcast_to(delay[None, None, :, :], (_N, _C, _DO, _DI))
    bf = jnp.floor(bd)
    bern = jax.random.bernoulli(jax.random.key(1), bd - bf)
    br0 = jnp.where(bern, bf + 1.0, bf).reshape(_M)
    return _delay_sc(input, br0)


# speedup vs baseline: 1.3168x; 1.0696x over previous
import functools

import jax
import jax.numpy as jnp
from jax import lax
from jax.experimental import pallas as pl
from jax.experimental.pallas import tpu as pltpu
from jax.experimental.pallas import tpu_sc as plsc

_T, _N, _C, _DO, _DI = 32, 16, 2, 512, 64
_M = _N * _C * _DO * _DI
_NW = 32
_COLS_W = _M // _NW
_NO = 8
_MC = _NO * _DI
_NCHUNK = _DO // _NO
_G = _MC // 16

_mesh = plsc.VectorSubcoreMesh(core_axis_name="c", subcore_axis_name="s")


def _sigmoid(v):
    return 1.0 / (1.0 + jnp.exp(-v))


@functools.partial(
    pl.kernel,
    mesh=_mesh,
    out_type=jax.ShapeDtypeStruct((_T, _N, _C, _DO, _DI), jnp.float32),
    scratch_types=[
        pltpu.VMEM((_T, _NO, _DI), jnp.float32),
        pltpu.VMEM((_T, _NO, _DI), jnp.float32),
        pltpu.VMEM((_MC,), jnp.float32),
        pltpu.SemaphoreType.DMA,
        pltpu.SemaphoreType.DMA,
    ],
    compiler_params=pltpu.CompilerParams(
        use_tc_tiling_on_sc=True, needs_layout_passes=False
    ),
)
def _delay_sc(x_hbm, br0_hbm, out_hbm, in_sp, out_sp, br_sp, sem_in, sem_out):
    wid = lax.axis_index("s") * 2 + lax.axis_index("c")
    n0 = wid // _C
    c0 = wid % _C

    def chunk_body(ci, carry):
        o0 = ci * _NO
        m0 = wid * _COLS_W + ci * _MC

        copies = [
            pltpu.make_async_copy(
                x_hbm.at[t, n0, c0, pl.ds(o0, _NO), :], in_sp.at[t], sem_in
            )
            for t in range(_T)
        ]
        cbr = pltpu.make_async_copy(br0_hbm.at[pl.ds(m0, _MC)], br_sp, sem_in)
        for cp in copies:
            cp.start()
        cbr.start()
        for cp in copies:
            cp.wait()
        cbr.wait()

        def group_body(j, c2):
            osub = j >> 2
            i0 = (j & 3) << 4
            ivec = i0 + lax.iota(jnp.int32, 16)
            ovec = jnp.full((16,), osub, jnp.int32)

            s0 = _sigmoid(in_sp[0, osub, pl.ds(i0, 16)])
            in_sp[0, osub, pl.ds(i0, 16)] = s0
            mx = s0
            am = jnp.zeros((16,), jnp.int32)
            for t in range(1, _T):
                st = _sigmoid(in_sp[t, osub, pl.ds(i0, 16)])
                in_sp[t, osub, pl.ds(i0, 16)] = st
                gt = st > mx
                am = jnp.where(gt, t, am)
                mx = jnp.where(gt, st, mx)

            brf = jnp.minimum(
                br_sp[pl.ds(j * 16, 16)], (31 - am).astype(jnp.float32)
            )
            br = brf.astype(jnp.int32)
            for t in range(_T):
                r = (t - br) & 31
                out_sp[t, osub, pl.ds(i0, 16)] = plsc.load_gather(
                    in_sp, [r, ovec, ivec]
                )
            return c2

        lax.fori_loop(0, _G, group_body, 0, unroll=False)

        ocopies = [
            pltpu.make_async_copy(
                out_sp.at[t], out_hbm.at[t, n0, c0, pl.ds(o0, _NO), :], sem_out
            )
            for t in range(_T)
        ]
        for cp in ocopies:
            cp.start()
        for cp in ocopies:
            cp.wait()
        return carry

    lax.fori_loop(0, _NCHUNK, chunk_body, 0, unroll=False)


def kernel(input, delay):
    bd = jnp.broadcast_to(delay[None, None, :, :], (_N, _C, _DO, _DI))
    bf = jnp.floor(bd)
    bern = jax.random.bernoulli(jax.random.key(1), bd - bf)
    br0 = jnp.where(bern, bf + 1.0, bf).reshape(_M)
    return _delay_sc(input, br0)


# speedup vs baseline: 2.1319x; 1.6190x over previous
import functools

import jax
import jax.numpy as jnp
from jax import lax
from jax.experimental import pallas as pl
from jax.experimental.pallas import tpu as pltpu
from jax.experimental.pallas import tpu_sc as plsc

_T, _N, _C, _DO, _DI = 32, 16, 2, 512, 64
_NW = 32
_TI = 8
_TO = 128
_NCHUNK = (_DI // _TI) * (_DO // _TO)
_OT = _DO // _TO
_G = (_TI * _TO) // 16

_mesh = plsc.VectorSubcoreMesh(core_axis_name="c", subcore_axis_name="s")


def _sigmoid(v):
    return 1.0 / (1.0 + jnp.exp(-v))


@functools.partial(
    pl.kernel,
    mesh=_mesh,
    out_type=jax.ShapeDtypeStruct((_T, _N, _C, _DI, _DO), jnp.float32),
    scratch_types=[
        pltpu.VMEM((_T, _TI, _TO), jnp.float32),
        pltpu.VMEM((_T, _TI, _TO), jnp.float32),
        pltpu.VMEM((_TI, _TO), jnp.float32),
        pltpu.SemaphoreType.DMA,
        pltpu.SemaphoreType.DMA,
    ],
    compiler_params=pltpu.CompilerParams(
        use_tc_tiling_on_sc=True, needs_layout_passes=False
    ),
)
def _delay_sc(x_hbm, br0_hbm, out_hbm, in_sp, out_sp, br_sp, sem_in, sem_out):
    wid = lax.axis_index("s") * 2 + lax.axis_index("c")
    n0 = wid // _C
    c0 = wid % _C

    def chunk_body(ci, carry):
        i0 = (ci // _OT) * _TI
        ob = (ci % _OT) * _TO

        copies = [
            pltpu.make_async_copy(
                x_hbm.at[t, n0, c0, pl.ds(i0, _TI), pl.ds(ob, _TO)],
                in_sp.at[t],
                sem_in,
            )
            for t in range(_T)
        ]
        cbr = pltpu.make_async_copy(
            br0_hbm.at[wid, pl.ds(i0, _TI), pl.ds(ob, _TO)], br_sp, sem_in
        )
        for cp in copies:
            cp.start()
        cbr.start()
        for cp in copies:
            cp.wait()
        cbr.wait()

        def group_body(j, c2):
            il = j >> 3
            o0 = (j & 7) << 4
            ovec = o0 + lax.iota(jnp.int32, 16)
            ivec = jnp.full((16,), il, jnp.int32)

            s0 = _sigmoid(in_sp[0, il, pl.ds(o0, 16)])
            in_sp[0, il, pl.ds(o0, 16)] = s0
            mx = s0
            am = jnp.zeros((16,), jnp.int32)
            for t in range(1, _T):
                st = _sigmoid(in_sp[t, il, pl.ds(o0, 16)])
                in_sp[t, il, pl.ds(o0, 16)] = st
                gt = st > mx
                am = jnp.where(gt, t, am)
                mx = jnp.where(gt, st, mx)

            brf = jnp.minimum(
                br_sp[il, pl.ds(o0, 16)], (31 - am).astype(jnp.float32)
            )
            br = brf.astype(jnp.int32)
            for t in range(_T):
                r = (t - br) & 31
                out_sp[t, il, pl.ds(o0, 16)] = plsc.load_gather(
                    in_sp, [r, ivec, ovec]
                )
            return c2

        lax.fori_loop(0, _G, group_body, 0, unroll=False)

        ocopies = [
            pltpu.make_async_copy(
                out_sp.at[t],
                out_hbm.at[t, n0, c0, pl.ds(i0, _TI), pl.ds(ob, _TO)],
                sem_out,
            )
            for t in range(_T)
        ]
        for cp in ocopies:
            cp.start()
        for cp in ocopies:
            cp.wait()
        return carry

    lax.fori_loop(0, _NCHUNK, chunk_body, 0, unroll=False)


def kernel(input, delay):
    bd = jnp.broadcast_to(delay[None, None, :, :], (_N, _C, _DO, _DI))
    bf = jnp.floor(bd)
    bern = jax.random.bernoulli(jax.random.key(1), bd - bf)
    br0 = jnp.where(bern, bf + 1.0, bf)
    br0_t = jnp.swapaxes(br0, 2, 3).reshape(_N * _C, _DI, _DO)
    x_t = jnp.swapaxes(input, 3, 4)
    out_t = _delay_sc(x_t, br0_t)
    return jnp.swapaxes(out_t, 3, 4)
